# baseline (speedup 1.0000x reference)
import functools

import jax
import jax.numpy as jnp
from jax import lax
from jax.experimental import pallas as pl
from jax.experimental.pallas import tpu as pltpu
from jax.experimental.pallas import tpu_sc as plsc

NC = 2
NS = 16
NW = NC * NS
EB = 128


def _worker_id():
    c = lax.axis_index("c")
    s = lax.axis_index("s")
    return c, s, s * NC + c


def _split_rows(n):
    span_a = ((n + NS - 1) // NS + 7) // 8 * 8
    span_last = n - span_a * (NS - 1)
    assert span_last > 0 and span_last % 8 == 0 and span_a % 8 == 0
    return span_a, span_last


def _per_tile_rows(s, n, fn):
    span_a, span_last = _split_rows(n)

    @pl.when(s < NS - 1)
    def _():
        fn(s * span_a, span_a)

    @pl.when(s == NS - 1)
    def _():
        fn((NS - 1) * span_a, span_last)


_DRING = 8
_DBLK = 8


def _sc_degree(dst2d, n_pad):
    nb_total, eba = dst2d.shape
    tpw = nb_total // NW
    nblk = tpw // _DBLK
    per_tile = n_pad // NS
    assert tpw * NW == nb_total and nblk * _DBLK == tpw and nblk % 2 == 0

    mesh = plsc.VectorSubcoreMesh(core_axis_name="c", subcore_axis_name="s")

    @functools.partial(
        pl.kernel,
        out_type=jax.ShapeDtypeStruct((NC * n_pad,), jnp.float32),
        mesh=mesh,
        scratch_types=[
            pltpu.VMEM((2, _DBLK, eba), jnp.int32),
            pltpu.VMEM((128,), jnp.float32),
            pltpu.VMEM((per_tile,), jnp.float32),
            pltpu.VMEM_SHARED((n_pad,), jnp.float32),
            pltpu.SemaphoreType.DMA((2,)),
            pltpu.SemaphoreType.DMA((_DRING,)),
        ],
    )
    def deg_kernel(dst_hbm, out_hbm, idx_v, ones_v, zbuf, acc, sem_i, sem_s):
        c, s, w = _worker_id()
        for j in range(128 // 16):
            ones_v[pl.ds(j * 16, 16)] = jnp.full((16,), 1.0, jnp.float32)
        for j in range(per_tile // 16):
            zbuf[pl.ds(j * 16, 16)] = jnp.zeros((16,), jnp.float32)

        def fire_idx(r, pb):
            pltpu.async_copy(dst_hbm.at[pl.ds(w * tpw + r * _DBLK, _DBLK)],
                             idx_v.at[pb], sem_i.at[pb])

        def drain_idx(pb):
            pltpu.make_async_copy(dst_hbm.at[pl.ds(0, _DBLK)], idx_v.at[pb],
                                  sem_i.at[pb]).wait()

        def fire_scatter(pb, row, j):
            pltpu.async_copy(ones_v.at[pl.ds(0, eba)],
                             acc.at[idx_v.at[pb].at[row]],
                             sem_s.at[j], add=True)

        def drain_scatter(j):
            pltpu.make_async_copy(ones_v.at[pl.ds(0, eba)],
                                  acc.at[idx_v.at[0].at[0]],
                                  sem_s.at[j]).wait()

        fire_idx(0, 0)
        pltpu.sync_copy(zbuf, acc.at[pl.ds(s * per_tile, per_tile)])
        plsc.subcore_barrier()

        def body(t, carry):
            for rr in range(2):
                r = t * 2 + rr
                for jj in range(_DBLK):
                    b = r * _DBLK + jj
                    if jj == 0:
                        drain_idx(rr)

                    @pl.when(b >= _DRING // 2)
                    def _():
                        drain_scatter((jj + _DRING // 2) % _DRING)

                    if jj == _DRING // 2:
                        @pl.when(r + 1 < nblk)
                        def _():
                            fire_idx(r + 1, 1 - rr)
                    fire_scatter(rr, jj, jj % _DRING)
            return carry

        lax.fori_loop(0, nblk // 2, body, 0)
        for k in range(_DRING // 2):
            drain_scatter((tpw - _DRING // 2 + k) % _DRING)
        plsc.subcore_barrier()
        pltpu.sync_copy(acc.at[pl.ds(s * per_tile, per_tile)],
                        out_hbm.at[pl.ds(c * n_pad + s * per_tile, per_tile)])

    return deg_kernel(dst2d)


_NBUF = 4
_LOOK = 2
_BLK = 8


def _sc_aggregate(g2h, src2d, dst2d):
    _, N, Dh = g2h.shape
    nb_total, eba = src2d.shape
    tpb = nb_total // NS
    nblk = tpb // _BLK
    assert tpb * NS == nb_total and nblk * _BLK == tpb and nblk % 2 == 0
    assert _BLK % _NBUF == 0 and _LOOK * 2 == _NBUF

    mesh = plsc.VectorSubcoreMesh(core_axis_name="c", subcore_axis_name="s")

    @functools.partial(
        pl.kernel,
        out_type=jax.ShapeDtypeStruct((NC, N, Dh), jnp.float32),
        mesh=mesh,
        scratch_types=[
            pltpu.VMEM((2, _BLK, eba), jnp.int32),
            pltpu.VMEM((2, _BLK, eba), jnp.int32),
            pltpu.VMEM((_NBUF, eba, Dh), jnp.float32),
            pltpu.VMEM_SHARED((N, Dh), jnp.float32),
            pltpu.SemaphoreType.DMA((_NBUF,)),
            pltpu.SemaphoreType.DMA((_NBUF,)),
            pltpu.SemaphoreType.DMA((2,)),
        ],
        compiler_params=pltpu.CompilerParams(use_tc_tiling_on_sc=False),
    )
    def agg_kernel(g_hbm, src_hbm, dst_hbm, out_hbm,
                   idx_s, idx_d, rows, acc, sem_g, sem_s, sem_i):
        c, s, _ = _worker_id()

        def on_half(fn):
            @pl.when(c == 0)
            def _():
                fn(g_hbm.at[0], out_hbm.at[0])

            @pl.when(c == 1)
            def _():
                fn(g_hbm.at[1], out_hbm.at[1])

        def fire_idx_block(r, pb):
            base = s * tpb + r * _BLK
            pltpu.async_copy(src_hbm.at[pl.ds(base, _BLK)], idx_s.at[pb],
                             sem_i.at[pb])
            pltpu.async_copy(dst_hbm.at[pl.ds(base, _BLK)], idx_d.at[pb],
                             sem_i.at[pb])

        def drain_idx_block(pb):
            pltpu.make_async_copy(src_hbm.at[pl.ds(0, _BLK)], idx_s.at[pb],
                                  sem_i.at[pb]).wait()
            pltpu.make_async_copy(dst_hbm.at[pl.ds(0, _BLK)], idx_d.at[pb],
                                  sem_i.at[pb]).wait()

        fire_idx_block(0, 0)

        def init(gh, oh):
            def cp(row0, span):
                pltpu.sync_copy(gh.at[pl.ds(row0, span)],
                                acc.at[pl.ds(row0, span)])
            _per_tile_rows(s, N, cp)

        on_half(init)
        drain_idx_block(0)
        plsc.subcore_barrier()

        def run_edges(gh, oh):
            def fire_gather(pb, row, j):
                pltpu.async_copy(gh.at[idx_s.at[pb].at[row]], rows.at[j],
                                 sem_g.at[j])

            def drain_gather(j):
                pltpu.make_async_copy(gh.at[idx_s.at[0].at[0]], rows.at[j],
                                      sem_g.at[j]).wait()

            def fire_scatter(pb, row, j):
                pltpu.async_copy(rows.at[j], acc.at[idx_d.at[pb].at[row]],
                                 sem_s.at[j], add=True)

            def drain_scatter(j):
                pltpu.make_async_copy(rows.at[j], acc.at[idx_d.at[0].at[0]],
                                      sem_s.at[j]).wait()

            for j in range(_LOOK):
                fire_gather(0, j, j)

            def body(t, carry):
                for rr in range(2):
                    r = t * 2 + rr
                    for jj in range(_BLK):
                        b = r * _BLK + jj
                        j = (b % _NBUF)
                        j2 = (jj + _LOOK) % _NBUF
                        drain_gather(j)
                        fire_scatter(rr, jj, j)

                        @pl.when(b >= _LOOK)
                        def _():
                            drain_scatter(j2)

                        if jj == _LOOK:
                            @pl.when(r + 1 < nblk)
                            def _():
                                fire_idx_block(r + 1, 1 - rr)
                        if jj == _BLK - _LOOK:
                            @pl.when(r + 1 < nblk)
                            def _():
                                drain_idx_block(1 - rr)
                        if jj < _BLK - _LOOK:
                            fire_gather(rr, jj + _LOOK, j2)
                        else:
                            @pl.when(b + _LOOK < tpb)
                            def _():
                                fire_gather(1 - rr, jj + _LOOK - _BLK, j2)
                return carry

            lax.fori_loop(0, nblk // 2, body, 0)
            for k in range(_LOOK):
                drain_scatter((tpb - _LOOK + k) % _NBUF)

        on_half(run_edges)
        plsc.subcore_barrier()

        def writeout(gh, oh):
            def cp(row0, span):
                pltpu.sync_copy(acc.at[pl.ds(row0, span)],
                                oh.at[pl.ds(row0, span)])
            _per_tile_rows(s, N, cp)

        on_half(writeout)

    return agg_kernel(g2h, src2d, dst2d)


_BN = 2000


def _tc_matmul(x, W1):
    N, D = x.shape
    H = W1.shape[0]

    def body(x_ref, w_ref, h_ref):
        h_ref[...] = lax.dot_general(x_ref[...], w_ref[...],
                                     (((1,), (1,)), ((), ())),
                                     preferred_element_type=jnp.float32)

    return pl.pallas_call(
        body,
        grid=(N // _BN,),
        in_specs=[
            pl.BlockSpec((_BN, D), lambda i: (i, 0)),
            pl.BlockSpec((H, D), lambda i: (0, 0)),
        ],
        out_specs=pl.BlockSpec((_BN, H), lambda i: (i, 0)),
        out_shape=jax.ShapeDtypeStruct((N, H), jnp.float32),
    )(x, W1)


def _tc_scale(h1, degp3):
    N, H = h1.shape
    Dh = H // 2

    def body(h_ref, dp_ref, dis_ref, g_ref):
        deg = dp_ref[0] + dp_ref[1] + 1.0
        dis = lax.rsqrt(deg)
        dis_ref[...] = dis
        v = h_ref[...] * dis
        g_ref[0] = v[:, :Dh]
        g_ref[1] = v[:, Dh:]

    return pl.pallas_call(
        body,
        grid=(N // _BN,),
        in_specs=[
            pl.BlockSpec((_BN, H), lambda i: (i, 0)),
            pl.BlockSpec((2, _BN, 1), lambda i: (0, i, 0)),
        ],
        out_specs=[
            pl.BlockSpec((_BN, 1), lambda i: (i, 0)),
            pl.BlockSpec((2, _BN, Dh), lambda i: (0, i, 0)),
        ],
        out_shape=[
            jax.ShapeDtypeStruct((N, 1), jnp.float32),
            jax.ShapeDtypeStruct((2, N, Dh), jnp.float32),
        ],
    )(h1, degp3)


def _tc_mid(p, dis, b, W):
    _, N, Dh = p.shape
    H = 2 * Dh
    Ho = W.shape[0]

    def body(p_ref, dis_ref, b_ref, w_ref, g_ref):
        agg = jnp.concatenate([p_ref[0], p_ref[1]], axis=1)
        z = jnp.maximum(dis_ref[...] * agg + b_ref[...], 0.0)
        h = lax.dot_general(z, w_ref[...], (((1,), (1,)), ((), ())),
                            preferred_element_type=jnp.float32)
        v = dis_ref[...] * h
        g_ref[0] = v[:, :Ho // 2]
        g_ref[1] = v[:, Ho // 2:]

    return pl.pallas_call(
        body,
        grid=(N // _BN,),
        in_specs=[
            pl.BlockSpec((2, _BN, Dh), lambda i: (0, i, 0)),
            pl.BlockSpec((_BN, 1), lambda i: (i, 0)),
            pl.BlockSpec((1, H), lambda i: (0, 0)),
            pl.BlockSpec((Ho, H), lambda i: (0, 0)),
        ],
        out_specs=pl.BlockSpec((2, _BN, Ho // 2), lambda i: (0, i, 0)),
        out_shape=jax.ShapeDtypeStruct((2, N, Ho // 2), jnp.float32),
    )(p, dis, b, W)


def _tc_final(q, dis, b, Wfc, bfc):
    _, N, Dh = q.shape
    H = 2 * Dh
    EMB = Wfc.shape[0]

    def body(q_ref, dis_ref, b_ref, w_ref, bfc_ref, out_ref):
        agg = jnp.concatenate([q_ref[0], q_ref[1]], axis=1)
        z = jnp.maximum(dis_ref[...] * agg + b_ref[...], 0.0)
        h = lax.dot_general(z, w_ref[...], (((1,), (1,)), ((), ())),
                            preferred_element_type=jnp.float32)
        out_ref[...] = h + bfc_ref[...]

    return pl.pallas_call(
        body,
        grid=(N // _BN,),
        in_specs=[
            pl.BlockSpec((2, _BN, Dh), lambda i: (0, i, 0)),
            pl.BlockSpec((_BN, 1), lambda i: (i, 0)),
            pl.BlockSpec((1, H), lambda i: (0, 0)),
            pl.BlockSpec((EMB, H), lambda i: (0, 0)),
            pl.BlockSpec((1, EMB), lambda i: (0, 0)),
        ],
        out_specs=pl.BlockSpec((_BN, EMB), lambda i: (i, 0)),
        out_shape=jax.ShapeDtypeStruct((N, EMB), jnp.float32),
    )(q, dis, b, Wfc, bfc)


def kernel(x, edge_index, W1, b1, W2, b2, Wfc, bfc):
    N, D = x.shape
    n_pad = 10240

    E = edge_index.shape[1]
    eba = 250
    src2d = edge_index[0].reshape(E // eba, eba)
    dst2d = edge_index[1].reshape(E // eba, eba)
    dst2d_deg = edge_index[1].reshape(E // 125, 125)

    degp = _sc_degree(dst2d_deg, n_pad)
    degp3 = degp.reshape(2, n_pad, 1)[:, :N]
    h1 = _tc_matmul(x, W1)
    dis, g1 = _tc_scale(h1, degp3)
    p = _sc_aggregate(g1, src2d, dst2d)
    g2 = _tc_mid(p, dis, b1.reshape(1, -1), W2)
    q = _sc_aggregate(g2, src2d, dst2d)
    out = _tc_final(q, dis, b2.reshape(1, -1), Wfc, bfc.reshape(1, -1))
    return out


# speedup vs baseline: 1.0512x; 1.0512x over previous
import functools

import jax
import jax.numpy as jnp
from jax import lax
from jax.experimental import pallas as pl
from jax.experimental.pallas import tpu as pltpu
from jax.experimental.pallas import tpu_sc as plsc

NC = 2
NS = 16
NW = NC * NS
EB = 128


def _worker_id():
    c = lax.axis_index("c")
    s = lax.axis_index("s")
    return c, s, s * NC + c


def _split_rows(n):
    span_a = ((n + NS - 1) // NS + 7) // 8 * 8
    span_last = n - span_a * (NS - 1)
    assert span_last > 0 and span_last % 8 == 0 and span_a % 8 == 0
    return span_a, span_last


def _per_tile_rows(s, n, fn):
    span_a, span_last = _split_rows(n)

    @pl.when(s < NS - 1)
    def _():
        fn(s * span_a, span_a)

    @pl.when(s == NS - 1)
    def _():
        fn((NS - 1) * span_a, span_last)


_DRING = 8
_DBLK = 8


def _sc_degree(dst2d, n_pad):
    nb_total, eba = dst2d.shape
    tpw = nb_total // NW
    nblk = tpw // _DBLK
    per_tile = n_pad // NS
    assert tpw * NW == nb_total and nblk * _DBLK == tpw and nblk % 2 == 0

    mesh = plsc.VectorSubcoreMesh(core_axis_name="c", subcore_axis_name="s")

    @functools.partial(
        pl.kernel,
        out_type=jax.ShapeDtypeStruct((NC * n_pad,), jnp.float32),
        mesh=mesh,
        scratch_types=[
            pltpu.VMEM((2, _DBLK, eba), jnp.int32),
            pltpu.VMEM((128,), jnp.float32),
            pltpu.VMEM((per_tile,), jnp.float32),
            pltpu.VMEM_SHARED((n_pad,), jnp.float32),
            pltpu.SemaphoreType.DMA((2,)),
            pltpu.SemaphoreType.DMA((_DRING,)),
        ],
    )
    def deg_kernel(dst_hbm, out_hbm, idx_v, ones_v, zbuf, acc, sem_i, sem_s):
        c, s, w = _worker_id()
        for j in range(128 // 16):
            ones_v[pl.ds(j * 16, 16)] = jnp.full((16,), 1.0, jnp.float32)
        for j in range(per_tile // 16):
            zbuf[pl.ds(j * 16, 16)] = jnp.zeros((16,), jnp.float32)

        def fire_idx(r, pb):
            pltpu.async_copy(dst_hbm.at[pl.ds(w * tpw + r * _DBLK, _DBLK)],
                             idx_v.at[pb], sem_i.at[pb])

        def drain_idx(pb):
            pltpu.make_async_copy(dst_hbm.at[pl.ds(0, _DBLK)], idx_v.at[pb],
                                  sem_i.at[pb]).wait()

        def fire_scatter(pb, row, j):
            pltpu.async_copy(ones_v.at[pl.ds(0, eba)],
                             acc.at[idx_v.at[pb].at[row]],
                             sem_s.at[j], add=True)

        def drain_scatter(j):
            pltpu.make_async_copy(ones_v.at[pl.ds(0, eba)],
                                  acc.at[idx_v.at[0].at[0]],
                                  sem_s.at[j]).wait()

        fire_idx(0, 0)
        pltpu.sync_copy(zbuf, acc.at[pl.ds(s * per_tile, per_tile)])
        plsc.subcore_barrier()

        def body(t, carry):
            for rr in range(2):
                r = t * 2 + rr
                for jj in range(_DBLK):
                    b = r * _DBLK + jj
                    if jj == 0:
                        drain_idx(rr)

                    @pl.when(b >= _DRING // 2)
                    def _():
                        drain_scatter((jj + _DRING // 2) % _DRING)

                    if jj == _DRING // 2:
                        @pl.when(r + 1 < nblk)
                        def _():
                            fire_idx(r + 1, 1 - rr)
                    fire_scatter(rr, jj, jj % _DRING)
            return carry

        lax.fori_loop(0, nblk // 2, body, 0)
        for k in range(_DRING // 2):
            drain_scatter((tpw - _DRING // 2 + k) % _DRING)
        plsc.subcore_barrier()
        pltpu.sync_copy(acc.at[pl.ds(s * per_tile, per_tile)],
                        out_hbm.at[pl.ds(c * n_pad + s * per_tile, per_tile)])

    return deg_kernel(dst2d)


_NBUF = 10
_LOOK = 5
_BLK = 20


def _sc_aggregate(g2h, src2d, dst2d):
    _, N, Dh = g2h.shape
    nb_total, eba = src2d.shape
    tpb = nb_total // NS
    nblk = tpb // _BLK
    assert tpb * NS == nb_total and nblk * _BLK == tpb and nblk % 2 == 0
    assert _BLK % _NBUF == 0 and _LOOK * 2 == _NBUF

    mesh = plsc.VectorSubcoreMesh(core_axis_name="c", subcore_axis_name="s")

    @functools.partial(
        pl.kernel,
        out_type=jax.ShapeDtypeStruct((NC, N, Dh), jnp.float32),
        mesh=mesh,
        scratch_types=[
            pltpu.VMEM((2, _BLK, eba), jnp.int32),
            pltpu.VMEM((2, _BLK, eba), jnp.int32),
            pltpu.VMEM((_NBUF, eba, Dh), jnp.float32),
            pltpu.VMEM_SHARED((N, Dh), jnp.float32),
            pltpu.SemaphoreType.DMA((_NBUF,)),
            pltpu.SemaphoreType.DMA((_NBUF,)),
            pltpu.SemaphoreType.DMA((2,)),
        ],
        compiler_params=pltpu.CompilerParams(use_tc_tiling_on_sc=False),
    )
    def agg_kernel(g_hbm, src_hbm, dst_hbm, out_hbm,
                   idx_s, idx_d, rows, acc, sem_g, sem_s, sem_i):
        c, s, _ = _worker_id()

        def on_half(fn):
            @pl.when(c == 0)
            def _():
                fn(g_hbm.at[0], out_hbm.at[0])

            @pl.when(c == 1)
            def _():
                fn(g_hbm.at[1], out_hbm.at[1])

        def fire_idx_block(r, pb):
            base = s * tpb + r * _BLK
            pltpu.async_copy(src_hbm.at[pl.ds(base, _BLK)], idx_s.at[pb],
                             sem_i.at[pb])
            pltpu.async_copy(dst_hbm.at[pl.ds(base, _BLK)], idx_d.at[pb],
                             sem_i.at[pb])

        def drain_idx_block(pb):
            pltpu.make_async_copy(src_hbm.at[pl.ds(0, _BLK)], idx_s.at[pb],
                                  sem_i.at[pb]).wait()
            pltpu.make_async_copy(dst_hbm.at[pl.ds(0, _BLK)], idx_d.at[pb],
                                  sem_i.at[pb]).wait()

        fire_idx_block(0, 0)

        def init(gh, oh):
            def cp(row0, span):
                pltpu.sync_copy(gh.at[pl.ds(row0, span)],
                                acc.at[pl.ds(row0, span)])
            _per_tile_rows(s, N, cp)

        on_half(init)
        drain_idx_block(0)
        plsc.subcore_barrier()

        def run_edges(gh, oh):
            def fire_gather(pb, row, j):
                pltpu.async_copy(gh.at[idx_s.at[pb].at[row]], rows.at[j],
                                 sem_g.at[j])

            def drain_gather(j):
                pltpu.make_async_copy(gh.at[idx_s.at[0].at[0]], rows.at[j],
                                      sem_g.at[j]).wait()

            def fire_scatter(pb, row, j):
                pltpu.async_copy(rows.at[j], acc.at[idx_d.at[pb].at[row]],
                                 sem_s.at[j], add=True)

            def drain_scatter(j):
                pltpu.make_async_copy(rows.at[j], acc.at[idx_d.at[0].at[0]],
                                      sem_s.at[j]).wait()

            for j in range(_LOOK):
                fire_gather(0, j, j)

            def body(t, carry):
                for rr in range(2):
                    r = t * 2 + rr
                    for jj in range(_BLK):
                        b = r * _BLK + jj
                        j = (b % _NBUF)
                        j2 = (jj + _LOOK) % _NBUF
                        drain_gather(j)
                        fire_scatter(rr, jj, j)

                        @pl.when(b >= _LOOK)
                        def _():
                            drain_scatter(j2)

                        if jj == _LOOK:
                            @pl.when(r + 1 < nblk)
                            def _():
                                fire_idx_block(r + 1, 1 - rr)
                        if jj == _BLK - _LOOK:
                            @pl.when(r + 1 < nblk)
                            def _():
                                drain_idx_block(1 - rr)
                        if jj < _BLK - _LOOK:
                            fire_gather(rr, jj + _LOOK, j2)
                        else:
                            @pl.when(b + _LOOK < tpb)
                            def _():
                                fire_gather(1 - rr, jj + _LOOK - _BLK, j2)
                return carry

            lax.fori_loop(0, nblk // 2, body, 0)
            for k in range(_LOOK):
                drain_scatter((tpb - _LOOK + k) % _NBUF)

        on_half(run_edges)
        plsc.subcore_barrier()

        def writeout(gh, oh):
            def cp(row0, span):
                pltpu.sync_copy(acc.at[pl.ds(row0, span)],
                                oh.at[pl.ds(row0, span)])
            _per_tile_rows(s, N, cp)

        on_half(writeout)

    return agg_kernel(g2h, src2d, dst2d)


_BN = 2000


def _tc_matmul(x, W1):
    N, D = x.shape
    H = W1.shape[0]

    def body(x_ref, w_ref, h_ref):
        h_ref[...] = lax.dot_general(x_ref[...], w_ref[...],
                                     (((1,), (1,)), ((), ())),
                                     preferred_element_type=jnp.float32)

    return pl.pallas_call(
        body,
        grid=(N // _BN,),
        in_specs=[
            pl.BlockSpec((_BN, D), lambda i: (i, 0)),
            pl.BlockSpec((H, D), lambda i: (0, 0)),
        ],
        out_specs=pl.BlockSpec((_BN, H), lambda i: (i, 0)),
        out_shape=jax.ShapeDtypeStruct((N, H), jnp.float32),
    )(x, W1)


def _tc_scale(h1, degp3):
    N, H = h1.shape
    Dh = H // 2

    def body(h_ref, dp_ref, dis_ref, g_ref):
        deg = dp_ref[0] + dp_ref[1] + 1.0
        dis = lax.rsqrt(deg)
        dis_ref[...] = dis
        v = h_ref[...] * dis
        g_ref[0] = v[:, :Dh]
        g_ref[1] = v[:, Dh:]

    return pl.pallas_call(
        body,
        grid=(N // _BN,),
        in_specs=[
            pl.BlockSpec((_BN, H), lambda i: (i, 0)),
            pl.BlockSpec((2, _BN, 1), lambda i: (0, i, 0)),
        ],
        out_specs=[
            pl.BlockSpec((_BN, 1), lambda i: (i, 0)),
            pl.BlockSpec((2, _BN, Dh), lambda i: (0, i, 0)),
        ],
        out_shape=[
            jax.ShapeDtypeStruct((N, 1), jnp.float32),
            jax.ShapeDtypeStruct((2, N, Dh), jnp.float32),
        ],
    )(h1, degp3)


def _tc_mid(p, dis, b, W):
    _, N, Dh = p.shape
    H = 2 * Dh
    Ho = W.shape[0]

    def body(p_ref, dis_ref, b_ref, w_ref, g_ref):
        agg = jnp.concatenate([p_ref[0], p_ref[1]], axis=1)
        z = jnp.maximum(dis_ref[...] * agg + b_ref[...], 0.0)
        h = lax.dot_general(z, w_ref[...], (((1,), (1,)), ((), ())),
                            preferred_element_type=jnp.float32)
        v = dis_ref[...] * h
        g_ref[0] = v[:, :Ho // 2]
        g_ref[1] = v[:, Ho // 2:]

    return pl.pallas_call(
        body,
        grid=(N // _BN,),
        in_specs=[
            pl.BlockSpec((2, _BN, Dh), lambda i: (0, i, 0)),
            pl.BlockSpec((_BN, 1), lambda i: (i, 0)),
            pl.BlockSpec((1, H), lambda i: (0, 0)),
            pl.BlockSpec((Ho, H), lambda i: (0, 0)),
        ],
        out_specs=pl.BlockSpec((2, _BN, Ho // 2), lambda i: (0, i, 0)),
        out_shape=jax.ShapeDtypeStruct((2, N, Ho // 2), jnp.float32),
    )(p, dis, b, W)


def _tc_final(q, dis, b, Wfc, bfc):
    _, N, Dh = q.shape
    H = 2 * Dh
    EMB = Wfc.shape[0]

    def body(q_ref, dis_ref, b_ref, w_ref, bfc_ref, out_ref):
        agg = jnp.concatenate([q_ref[0], q_ref[1]], axis=1)
        z = jnp.maximum(dis_ref[...] * agg + b_ref[...], 0.0)
        h = lax.dot_general(z, w_ref[...], (((1,), (1,)), ((), ())),
                            preferred_element_type=jnp.float32)
        out_ref[...] = h + bfc_ref[...]

    return pl.pallas_call(
        body,
        grid=(N // _BN,),
        in_specs=[
            pl.BlockSpec((2, _BN, Dh), lambda i: (0, i, 0)),
            pl.BlockSpec((_BN, 1), lambda i: (i, 0)),
            pl.BlockSpec((1, H), lambda i: (0, 0)),
            pl.BlockSpec((EMB, H), lambda i: (0, 0)),
            pl.BlockSpec((1, EMB), lambda i: (0, 0)),
        ],
        out_specs=pl.BlockSpec((_BN, EMB), lambda i: (i, 0)),
        out_shape=jax.ShapeDtypeStruct((N, EMB), jnp.float32),
    )(q, dis, b, Wfc, bfc)


def kernel(x, edge_index, W1, b1, W2, b2, Wfc, bfc):
    N, D = x.shape
    n_pad = 10240

    E = edge_index.shape[1]
    eba = 125
    src2d = edge_index[0].reshape(E // eba, eba)
    dst2d = edge_index[1].reshape(E // eba, eba)
    dst2d_deg = edge_index[1].reshape(E // 125, 125)

    degp = _sc_degree(dst2d_deg, n_pad)
    degp3 = degp.reshape(2, n_pad, 1)[:, :N]
    h1 = _tc_matmul(x, W1)
    dis, g1 = _tc_scale(h1, degp3)
    p = _sc_aggregate(g1, src2d, dst2d)
    g2 = _tc_mid(p, dis, b1.reshape(1, -1), W2)
    q = _sc_aggregate(g2, src2d, dst2d)
    out = _tc_final(q, dis, b2.reshape(1, -1), Wfc, bfc.reshape(1, -1))
    return out


# speedup vs baseline: 1.0744x; 1.0220x over previous
import functools

import jax
import jax.numpy as jnp
from jax import lax
from jax.experimental import pallas as pl
from jax.experimental.pallas import tpu as pltpu
from jax.experimental.pallas import tpu_sc as plsc

NC = 2
NS = 16
NW = NC * NS
EB = 128


def _worker_id():
    c = lax.axis_index("c")
    s = lax.axis_index("s")
    return c, s, s * NC + c


def _split_rows(n):
    span_a = ((n + NS - 1) // NS + 7) // 8 * 8
    span_last = n - span_a * (NS - 1)
    assert span_last > 0 and span_last % 8 == 0 and span_a % 8 == 0
    return span_a, span_last


def _per_tile_rows(s, n, fn):
    span_a, span_last = _split_rows(n)

    @pl.when(s < NS - 1)
    def _():
        fn(s * span_a, span_a)

    @pl.when(s == NS - 1)
    def _():
        fn((NS - 1) * span_a, span_last)


_DRING = 8
_DBLK = 8


def _sc_degree(dst2d, n_pad):
    nb_total, eba = dst2d.shape
    tpw = nb_total // NW
    nblk = tpw // _DBLK
    per_tile = n_pad // NS
    assert tpw * NW == nb_total and nblk * _DBLK == tpw and nblk % 2 == 0

    mesh = plsc.VectorSubcoreMesh(core_axis_name="c", subcore_axis_name="s")

    @functools.partial(
        pl.kernel,
        out_type=jax.ShapeDtypeStruct((NC * n_pad,), jnp.float32),
        mesh=mesh,
        scratch_types=[
            pltpu.VMEM((2, _DBLK, eba), jnp.int32),
            pltpu.VMEM((128,), jnp.float32),
            pltpu.VMEM((per_tile,), jnp.float32),
            pltpu.VMEM_SHARED((n_pad,), jnp.float32),
            pltpu.SemaphoreType.DMA((2,)),
            pltpu.SemaphoreType.DMA((_DRING,)),
        ],
    )
    def deg_kernel(dst_hbm, out_hbm, idx_v, ones_v, zbuf, acc, sem_i, sem_s):
        c, s, w = _worker_id()
        for j in range(128 // 16):
            ones_v[pl.ds(j * 16, 16)] = jnp.full((16,), 1.0, jnp.float32)
        for j in range(per_tile // 16):
            zbuf[pl.ds(j * 16, 16)] = jnp.zeros((16,), jnp.float32)

        def fire_idx(r, pb):
            pltpu.async_copy(dst_hbm.at[pl.ds(w * tpw + r * _DBLK, _DBLK)],
                             idx_v.at[pb], sem_i.at[pb])

        def drain_idx(pb):
            pltpu.make_async_copy(dst_hbm.at[pl.ds(0, _DBLK)], idx_v.at[pb],
                                  sem_i.at[pb]).wait()

        def fire_scatter(pb, row, j):
            pltpu.async_copy(ones_v.at[pl.ds(0, eba)],
                             acc.at[idx_v.at[pb].at[row]],
                             sem_s.at[j], add=True)

        def drain_scatter(j):
            pltpu.make_async_copy(ones_v.at[pl.ds(0, eba)],
                                  acc.at[idx_v.at[0].at[0]],
                                  sem_s.at[j]).wait()

        fire_idx(0, 0)
        pltpu.sync_copy(zbuf, acc.at[pl.ds(s * per_tile, per_tile)])
        plsc.subcore_barrier()

        def body(t, carry):
            for rr in range(2):
                r = t * 2 + rr
                for jj in range(_DBLK):
                    b = r * _DBLK + jj
                    if jj == 0:
                        drain_idx(rr)

                    @pl.when(b >= _DRING // 2)
                    def _():
                        drain_scatter((jj + _DRING // 2) % _DRING)

                    if jj == _DRING // 2:
                        @pl.when(r + 1 < nblk)
                        def _():
                            fire_idx(r + 1, 1 - rr)
                    fire_scatter(rr, jj, jj % _DRING)
            return carry

        lax.fori_loop(0, nblk // 2, body, 0)
        for k in range(_DRING // 2):
            drain_scatter((tpw - _DRING // 2 + k) % _DRING)
        plsc.subcore_barrier()
        pltpu.sync_copy(acc.at[pl.ds(s * per_tile, per_tile)],
                        out_hbm.at[pl.ds(c * n_pad + s * per_tile, per_tile)])

    return deg_kernel(dst2d)


_NBUF = 10
_LOOK = 5
_BLK = 20


def _sc_aggregate(g2h, src2d, dst2d):
    _, N, Dh = g2h.shape
    nb_total, eba = src2d.shape
    tpb = nb_total // NS
    nblk = tpb // _BLK
    assert tpb * NS == nb_total and nblk * _BLK == tpb and nblk % 2 == 0
    assert _BLK % _NBUF == 0 and _LOOK * 2 == _NBUF

    mesh = plsc.VectorSubcoreMesh(core_axis_name="c", subcore_axis_name="s")

    @functools.partial(
        pl.kernel,
        out_type=jax.ShapeDtypeStruct((NC, N, Dh), jnp.float32),
        mesh=mesh,
        scratch_types=[
            pltpu.VMEM((2, _BLK, eba), jnp.int32),
            pltpu.VMEM((2, _BLK, eba), jnp.int32),
            pltpu.VMEM((_NBUF, eba, Dh), jnp.float32),
            pltpu.VMEM_SHARED((N, Dh), jnp.float32),
            pltpu.SemaphoreType.DMA((_NBUF,)),
            pltpu.SemaphoreType.DMA((_NBUF,)),
            pltpu.SemaphoreType.DMA((2,)),
        ],
        compiler_params=pltpu.CompilerParams(use_tc_tiling_on_sc=False),
    )
    def agg_kernel(g_hbm, src_hbm, dst_hbm, out_hbm,
                   idx_s, idx_d, rows, acc, sem_g, sem_s, sem_i):
        c, s, _ = _worker_id()

        def on_half(fn):
            @pl.when(c == 0)
            def _():
                fn(g_hbm.at[0], out_hbm.at[0])

            @pl.when(c == 1)
            def _():
                fn(g_hbm.at[1], out_hbm.at[1])

        def fire_idx_block(r, pb):
            base = s * tpb + r * _BLK
            pltpu.async_copy(src_hbm.at[pl.ds(base, _BLK)], idx_s.at[pb],
                             sem_i.at[pb])
            pltpu.async_copy(dst_hbm.at[pl.ds(base, _BLK)], idx_d.at[pb],
                             sem_i.at[pb])

        def drain_idx_block(pb):
            pltpu.make_async_copy(src_hbm.at[pl.ds(0, _BLK)], idx_s.at[pb],
                                  sem_i.at[pb]).wait()
            pltpu.make_async_copy(dst_hbm.at[pl.ds(0, _BLK)], idx_d.at[pb],
                                  sem_i.at[pb]).wait()

        fire_idx_block(0, 0)

        def init(gh, oh):
            def cp(row0, span):
                pltpu.sync_copy(gh.at[pl.ds(row0, span)],
                                acc.at[pl.ds(row0, span)])
            _per_tile_rows(s, N, cp)

        on_half(init)
        drain_idx_block(0)
        plsc.subcore_barrier()

        def run_edges(gh, oh):
            def fire_gather(pb, row, j):
                pltpu.async_copy(gh.at[idx_s.at[pb].at[row]], rows.at[j],
                                 sem_g.at[j])

            def drain_gather(j):
                pltpu.make_async_copy(gh.at[idx_s.at[0].at[0]], rows.at[j],
                                      sem_g.at[j]).wait()

            def fire_scatter(pb, row, j):
                pltpu.async_copy(rows.at[j], acc.at[idx_d.at[pb].at[row]],
                                 sem_s.at[j], add=True)

            def drain_scatter(j):
                pltpu.make_async_copy(rows.at[j], acc.at[idx_d.at[0].at[0]],
                                      sem_s.at[j]).wait()

            for j in range(_LOOK):
                fire_gather(0, j, j)

            def body(t, carry):
                for rr in range(2):
                    r = t * 2 + rr
                    for jj in range(_BLK):
                        b = r * _BLK + jj
                        j = (b % _NBUF)
                        j2 = (jj + _LOOK) % _NBUF
                        drain_gather(j)
                        fire_scatter(rr, jj, j)

                        @pl.when(b >= _LOOK)
                        def _():
                            drain_scatter(j2)

                        if jj == _LOOK:
                            @pl.when(r + 1 < nblk)
                            def _():
                                fire_idx_block(r + 1, 1 - rr)
                        if jj == _BLK - _LOOK:
                            @pl.when(r + 1 < nblk)
                            def _():
                                drain_idx_block(1 - rr)
                        if jj < _BLK - _LOOK:
                            fire_gather(rr, jj + _LOOK, j2)
                        else:
                            @pl.when(b + _LOOK < tpb)
                            def _():
                                fire_gather(1 - rr, jj + _LOOK - _BLK, j2)
                return carry

            lax.fori_loop(0, nblk // 2, body, 0)
            for k in range(_LOOK):
                drain_scatter((tpb - _LOOK + k) % _NBUF)

        on_half(run_edges)
        plsc.subcore_barrier()

        def writeout(gh, oh):
            def cp(row0, span):
                pltpu.sync_copy(acc.at[pl.ds(row0, span)],
                                oh.at[pl.ds(row0, span)])
            _per_tile_rows(s, N, cp)

        on_half(writeout)

    return agg_kernel(g2h, src2d, dst2d)


_BN = 2000


def _tc_matmul(x, W1):
    N, D = x.shape
    H = W1.shape[0]

    def body(x_ref, w_ref, h_ref):
        h_ref[...] = lax.dot_general(x_ref[...], w_ref[...],
                                     (((1,), (1,)), ((), ())),
                                     preferred_element_type=jnp.float32)

    return pl.pallas_call(
        body,
        grid=(N // _BN,),
        in_specs=[
            pl.BlockSpec((_BN, D), lambda i: (i, 0)),
            pl.BlockSpec((H, D), lambda i: (0, 0)),
        ],
        out_specs=pl.BlockSpec((_BN, H), lambda i: (i, 0)),
        out_shape=jax.ShapeDtypeStruct((N, H), jnp.float32),
    )(x, W1)


def _tc_scale(h1, degp3):
    N, H = h1.shape
    Dh = H // 2

    def body(h_ref, dp_ref, dis_ref, g_ref):
        deg = dp_ref[0] + dp_ref[1] + 1.0
        dis = lax.rsqrt(deg)
        dis_ref[...] = dis
        v = h_ref[...] * dis
        g_ref[0] = v[:, :Dh]
        g_ref[1] = v[:, Dh:]

    return pl.pallas_call(
        body,
        grid=(N // _BN,),
        in_specs=[
            pl.BlockSpec((_BN, H), lambda i: (i, 0)),
            pl.BlockSpec((2, _BN, 1), lambda i: (0, i, 0)),
        ],
        out_specs=[
            pl.BlockSpec((_BN, 1), lambda i: (i, 0)),
            pl.BlockSpec((2, _BN, Dh), lambda i: (0, i, 0)),
        ],
        out_shape=[
            jax.ShapeDtypeStruct((N, 1), jnp.float32),
            jax.ShapeDtypeStruct((2, N, Dh), jnp.float32),
        ],
    )(h1, degp3)


def _tc_mid(p, dis, b, W):
    _, N, Dh = p.shape
    H = 2 * Dh
    Ho = W.shape[0]

    def body(p_ref, dis_ref, b_ref, w_ref, g_ref):
        agg = jnp.concatenate([p_ref[0], p_ref[1]], axis=1)
        z = jnp.maximum(dis_ref[...] * agg + b_ref[...], 0.0)
        h = lax.dot_general(z, w_ref[...], (((1,), (1,)), ((), ())),
                            preferred_element_type=jnp.float32)
        v = dis_ref[...] * h
        g_ref[0] = v[:, :Ho // 2]
        g_ref[1] = v[:, Ho // 2:]

    return pl.pallas_call(
        body,
        grid=(N // _BN,),
        in_specs=[
            pl.BlockSpec((2, _BN, Dh), lambda i: (0, i, 0)),
            pl.BlockSpec((_BN, 1), lambda i: (i, 0)),
            pl.BlockSpec((1, H), lambda i: (0, 0)),
            pl.BlockSpec((Ho, H), lambda i: (0, 0)),
        ],
        out_specs=pl.BlockSpec((2, _BN, Ho // 2), lambda i: (0, i, 0)),
        out_shape=jax.ShapeDtypeStruct((2, N, Ho // 2), jnp.float32),
    )(p, dis, b, W)


def _tc_final(q, dis, b, Wfc, bfc):
    _, N, Dh = q.shape
    H = 2 * Dh
    EMB = Wfc.shape[0]

    def body(q_ref, dis_ref, b_ref, w_ref, bfc_ref, out_ref):
        agg = jnp.concatenate([q_ref[0], q_ref[1]], axis=1)
        z = jnp.maximum(dis_ref[...] * agg + b_ref[...], 0.0)
        h = lax.dot_general(z, w_ref[...], (((1,), (1,)), ((), ())),
                            preferred_element_type=jnp.float32)
        out_ref[...] = h + bfc_ref[...]

    return pl.pallas_call(
        body,
        grid=(N // _BN,),
        in_specs=[
            pl.BlockSpec((2, _BN, Dh), lambda i: (0, i, 0)),
            pl.BlockSpec((_BN, 1), lambda i: (i, 0)),
            pl.BlockSpec((1, H), lambda i: (0, 0)),
            pl.BlockSpec((EMB, H), lambda i: (0, 0)),
            pl.BlockSpec((1, EMB), lambda i: (0, 0)),
        ],
        out_specs=pl.BlockSpec((_BN, EMB), lambda i: (i, 0)),
        out_shape=jax.ShapeDtypeStruct((N, EMB), jnp.float32),
    )(q, dis, b, Wfc, bfc)


def kernel(x, edge_index, W1, b1, W2, b2, Wfc, bfc):
    N, D = x.shape
    n_pad = 10240

    E = edge_index.shape[1]
    eba = 125
    src2d = edge_index[0].reshape(E // eba, eba)
    dst2d = edge_index[1].reshape(E // eba, eba)
    dst2d_deg = edge_index[1].reshape(E // 125, 125)

    degp = _sc_degree(dst2d_deg, n_pad)
    degp3 = degp.reshape(2, n_pad, 1)
    h1 = _tc_matmul(x, W1)
    dis, g1 = _tc_scale(h1, degp3)
    p = _sc_aggregate(g1, src2d, dst2d)
    g2 = _tc_mid(p, dis, b1.reshape(1, -1), W2)
    q = _sc_aggregate(g2, src2d, dst2d)
    out = _tc_final(q, dis, b2.reshape(1, -1), Wfc, bfc.reshape(1, -1))
    return out


# speedup vs baseline: 1.0850x; 1.0099x over previous
import functools

import jax
import jax.numpy as jnp
from jax import lax
from jax.experimental import pallas as pl
from jax.experimental.pallas import tpu as pltpu
from jax.experimental.pallas import tpu_sc as plsc

NC = 2
NS = 16
NW = NC * NS
EB = 128


def _worker_id():
    c = lax.axis_index("c")
    s = lax.axis_index("s")
    return c, s, s * NC + c


def _split_rows(n):
    span_a = ((n + NS - 1) // NS + 7) // 8 * 8
    span_last = n - span_a * (NS - 1)
    assert span_last > 0 and span_last % 8 == 0 and span_a % 8 == 0
    return span_a, span_last


def _per_tile_rows(s, n, fn):
    span_a, span_last = _split_rows(n)

    @pl.when(s < NS - 1)
    def _():
        fn(s * span_a, span_a)

    @pl.when(s == NS - 1)
    def _():
        fn((NS - 1) * span_a, span_last)


_DRING = 8
_DBLK = 8


def _sc_degree(dst2d, n_pad):
    nb_total, eba = dst2d.shape
    tpw = nb_total // NW
    nblk = tpw // _DBLK
    per_tile = n_pad // NS
    assert tpw * NW == nb_total and nblk * _DBLK == tpw and nblk % 2 == 0

    mesh = plsc.VectorSubcoreMesh(core_axis_name="c", subcore_axis_name="s")

    @functools.partial(
        pl.kernel,
        out_type=jax.ShapeDtypeStruct((NC * n_pad,), jnp.float32),
        mesh=mesh,
        scratch_types=[
            pltpu.VMEM((2, _DBLK, eba), jnp.int32),
            pltpu.VMEM((128,), jnp.float32),
            pltpu.VMEM((per_tile,), jnp.float32),
            pltpu.VMEM_SHARED((n_pad,), jnp.float32),
            pltpu.SemaphoreType.DMA((2,)),
            pltpu.SemaphoreType.DMA((_DRING,)),
        ],
    )
    def deg_kernel(dst_hbm, out_hbm, idx_v, ones_v, zbuf, acc, sem_i, sem_s):
        c, s, w = _worker_id()
        for j in range(128 // 16):
            ones_v[pl.ds(j * 16, 16)] = jnp.full((16,), 1.0, jnp.float32)
        for j in range(per_tile // 16):
            zbuf[pl.ds(j * 16, 16)] = jnp.zeros((16,), jnp.float32)

        def fire_idx(r, pb):
            pltpu.async_copy(dst_hbm.at[pl.ds(w * tpw + r * _DBLK, _DBLK)],
                             idx_v.at[pb], sem_i.at[pb])

        def drain_idx(pb):
            pltpu.make_async_copy(dst_hbm.at[pl.ds(0, _DBLK)], idx_v.at[pb],
                                  sem_i.at[pb]).wait()

        def fire_scatter(pb, row, j):
            pltpu.async_copy(ones_v.at[pl.ds(0, eba)],
                             acc.at[idx_v.at[pb].at[row]],
                             sem_s.at[j], add=True)

        def drain_scatter(j):
            pltpu.make_async_copy(ones_v.at[pl.ds(0, eba)],
                                  acc.at[idx_v.at[0].at[0]],
                                  sem_s.at[j]).wait()

        fire_idx(0, 0)
        pltpu.sync_copy(zbuf, acc.at[pl.ds(s * per_tile, per_tile)])
        plsc.subcore_barrier()

        def body(t, carry):
            for rr in range(2):
                r = t * 2 + rr
                for jj in range(_DBLK):
                    b = r * _DBLK + jj
                    if jj == 0:
                        drain_idx(rr)

                    @pl.when(b >= _DRING // 2)
                    def _():
                        drain_scatter((jj + _DRING // 2) % _DRING)

                    if jj == _DRING // 2:
                        @pl.when(r + 1 < nblk)
                        def _():
                            fire_idx(r + 1, 1 - rr)
                    fire_scatter(rr, jj, jj % _DRING)
            return carry

        lax.fori_loop(0, nblk // 2, body, 0)
        for k in range(_DRING // 2):
            drain_scatter((tpw - _DRING // 2 + k) % _DRING)
        plsc.subcore_barrier()
        pltpu.sync_copy(acc.at[pl.ds(s * per_tile, per_tile)],
                        out_hbm.at[pl.ds(c * n_pad + s * per_tile, per_tile)])

    return deg_kernel(dst2d)


_NBUF = 10
_LOOK = 5
_BLK = 20


def _sc_aggregate(g2h, src2d, dst2d):
    _, N, Dh = g2h.shape
    nb_total, eba = src2d.shape
    tpb = nb_total // NS
    nblk = tpb // _BLK
    assert tpb * NS == nb_total and nblk * _BLK == tpb and nblk % 2 == 0
    assert _BLK % _NBUF == 0 and _LOOK * 2 == _NBUF

    mesh = plsc.VectorSubcoreMesh(core_axis_name="c", subcore_axis_name="s")

    @functools.partial(
        pl.kernel,
        out_type=jax.ShapeDtypeStruct((NC, N, Dh), jnp.float32),
        mesh=mesh,
        scratch_types=[
            pltpu.VMEM((2, _BLK, eba), jnp.int32),
            pltpu.VMEM((2, _BLK, eba), jnp.int32),
            pltpu.VMEM((_NBUF, eba, Dh), jnp.float32),
            pltpu.VMEM_SHARED((N, Dh), jnp.float32),
            pltpu.SemaphoreType.DMA((_NBUF,)),
            pltpu.SemaphoreType.DMA((_NBUF,)),
            pltpu.SemaphoreType.DMA((2,)),
        ],
        compiler_params=pltpu.CompilerParams(use_tc_tiling_on_sc=False),
    )
    def agg_kernel(g_hbm, src_hbm, dst_hbm, out_hbm,
                   idx_s, idx_d, rows, acc, sem_g, sem_s, sem_i):
        c, s, _ = _worker_id()

        def on_half(fn):
            @pl.when(c == 0)
            def _():
                fn(g_hbm.at[0], out_hbm.at[0])

            @pl.when(c == 1)
            def _():
                fn(g_hbm.at[1], out_hbm.at[1])

        def fire_idx_block(r, pb):
            base = s * tpb + r * _BLK
            pltpu.async_copy(src_hbm.at[pl.ds(base, _BLK)], idx_s.at[pb],
                             sem_i.at[pb])
            pltpu.async_copy(dst_hbm.at[pl.ds(base, _BLK)], idx_d.at[pb],
                             sem_i.at[pb])

        def drain_idx_block(pb):
            pltpu.make_async_copy(src_hbm.at[pl.ds(0, _BLK)], idx_s.at[pb],
                                  sem_i.at[pb]).wait()
            pltpu.make_async_copy(dst_hbm.at[pl.ds(0, _BLK)], idx_d.at[pb],
                                  sem_i.at[pb]).wait()

        fire_idx_block(0, 0)

        def init(gh, oh):
            def cp(row0, span):
                pltpu.sync_copy(gh.at[pl.ds(row0, span)],
                                acc.at[pl.ds(row0, span)])
            _per_tile_rows(s, N, cp)

        on_half(init)
        drain_idx_block(0)
        plsc.subcore_barrier()

        def run_edges(gh, oh):
            def fire_gather(pb, row, j):
                pltpu.async_copy(gh.at[idx_s.at[pb].at[row]], rows.at[j],
                                 sem_g.at[j])

            def drain_gather(j):
                pltpu.make_async_copy(gh.at[idx_s.at[0].at[0]], rows.at[j],
                                      sem_g.at[j]).wait()

            def fire_scatter(pb, row, j):
                pltpu.async_copy(rows.at[j], acc.at[idx_d.at[pb].at[row]],
                                 sem_s.at[j], add=True)

            def drain_scatter(j):
                pltpu.make_async_copy(rows.at[j], acc.at[idx_d.at[0].at[0]],
                                      sem_s.at[j]).wait()

            for j in range(_LOOK):
                fire_gather(0, j, j)

            def body(t, carry):
                for rr in range(2):
                    r = t * 2 + rr
                    for jj in range(_BLK):
                        b = r * _BLK + jj
                        j = (b % _NBUF)
                        j2 = (jj + _LOOK) % _NBUF
                        drain_gather(j)
                        fire_scatter(rr, jj, j)

                        @pl.when(b >= _LOOK)
                        def _():
                            drain_scatter(j2)

                        if jj == _LOOK:
                            @pl.when(r + 1 < nblk)
                            def _():
                                fire_idx_block(r + 1, 1 - rr)
                        if jj == _BLK - _LOOK:
                            @pl.when(r + 1 < nblk)
                            def _():
                                drain_idx_block(1 - rr)
                        if jj < _BLK - _LOOK:
                            fire_gather(rr, jj + _LOOK, j2)
                        else:
                            @pl.when(b + _LOOK < tpb)
                            def _():
                                fire_gather(1 - rr, jj + _LOOK - _BLK, j2)
                return carry

            lax.fori_loop(0, nblk // 2, body, 0)
            for k in range(_LOOK):
                drain_scatter((tpb - _LOOK + k) % _NBUF)

        on_half(run_edges)
        plsc.subcore_barrier()

        def writeout(gh, oh):
            def cp(row0, span):
                pltpu.sync_copy(acc.at[pl.ds(row0, span)],
                                oh.at[pl.ds(row0, span)])
            _per_tile_rows(s, N, cp)

        on_half(writeout)

    return agg_kernel(g2h, src2d, dst2d)


_BN = 2000


def _tc_prologue(x, W1, degp3):
    N, D = x.shape
    H = W1.shape[0]
    Dh = H // 2

    def body(x_ref, w_ref, dp_ref, dis_ref, g_ref):
        deg = dp_ref[0] + dp_ref[1] + 1.0
        dis = lax.rsqrt(deg)
        dis_ref[...] = dis
        h = lax.dot_general(x_ref[...], w_ref[...],
                            (((1,), (1,)), ((), ())),
                            preferred_element_type=jnp.float32)
        v = h * dis
        g_ref[0] = v[:, :Dh]
        g_ref[1] = v[:, Dh:]

    return pl.pallas_call(
        body,
        grid=(N // _BN,),
        in_specs=[
            pl.BlockSpec((_BN, D), lambda i: (i, 0)),
            pl.BlockSpec((H, D), lambda i: (0, 0)),
            pl.BlockSpec((2, _BN, 1), lambda i: (0, i, 0)),
        ],
        out_specs=[
            pl.BlockSpec((_BN, 1), lambda i: (i, 0)),
            pl.BlockSpec((2, _BN, Dh), lambda i: (0, i, 0)),
        ],
        out_shape=[
            jax.ShapeDtypeStruct((N, 1), jnp.float32),
            jax.ShapeDtypeStruct((2, N, Dh), jnp.float32),
        ],
    )(x, W1, degp3)


def _tc_mid(p, dis, b, W):
    _, N, Dh = p.shape
    H = 2 * Dh
    Ho = W.shape[0]

    def body(p_ref, dis_ref, b_ref, w_ref, g_ref):
        agg = jnp.concatenate([p_ref[0], p_ref[1]], axis=1)
        z = jnp.maximum(dis_ref[...] * agg + b_ref[...], 0.0)
        h = lax.dot_general(z, w_ref[...], (((1,), (1,)), ((), ())),
                            preferred_element_type=jnp.float32)
        v = dis_ref[...] * h
        g_ref[0] = v[:, :Ho // 2]
        g_ref[1] = v[:, Ho // 2:]

    return pl.pallas_call(
        body,
        grid=(N // _BN,),
        in_specs=[
            pl.BlockSpec((2, _BN, Dh), lambda i: (0, i, 0)),
            pl.BlockSpec((_BN, 1), lambda i: (i, 0)),
            pl.BlockSpec((1, H), lambda i: (0, 0)),
            pl.BlockSpec((Ho, H), lambda i: (0, 0)),
        ],
        out_specs=pl.BlockSpec((2, _BN, Ho // 2), lambda i: (0, i, 0)),
        out_shape=jax.ShapeDtypeStruct((2, N, Ho // 2), jnp.float32),
    )(p, dis, b, W)


def _tc_final(q, dis, b, Wfc, bfc):
    _, N, Dh = q.shape
    H = 2 * Dh
    EMB = Wfc.shape[0]

    def body(q_ref, dis_ref, b_ref, w_ref, bfc_ref, out_ref):
        agg = jnp.concatenate([q_ref[0], q_ref[1]], axis=1)
        z = jnp.maximum(dis_ref[...] * agg + b_ref[...], 0.0)
        h = lax.dot_general(z, w_ref[...], (((1,), (1,)), ((), ())),
                            preferred_element_type=jnp.float32)
        out_ref[...] = h + bfc_ref[...]

    return pl.pallas_call(
        body,
        grid=(N // _BN,),
        in_specs=[
            pl.BlockSpec((2, _BN, Dh), lambda i: (0, i, 0)),
            pl.BlockSpec((_BN, 1), lambda i: (i, 0)),
            pl.BlockSpec((1, H), lambda i: (0, 0)),
            pl.BlockSpec((EMB, H), lambda i: (0, 0)),
            pl.BlockSpec((1, EMB), lambda i: (0, 0)),
        ],
        out_specs=pl.BlockSpec((_BN, EMB), lambda i: (i, 0)),
        out_shape=jax.ShapeDtypeStruct((N, EMB), jnp.float32),
    )(q, dis, b, Wfc, bfc)


def kernel(x, edge_index, W1, b1, W2, b2, Wfc, bfc):
    N, D = x.shape
    n_pad = 10240

    E = edge_index.shape[1]
    eba = 125
    src2d = edge_index[0].reshape(E // eba, eba)
    dst2d = edge_index[1].reshape(E // eba, eba)
    dst2d_deg = edge_index[1].reshape(E // 125, 125)

    degp = _sc_degree(dst2d_deg, n_pad)
    degp3 = degp.reshape(2, n_pad, 1)
    dis, g1 = _tc_prologue(x, W1, degp3)
    p = _sc_aggregate(g1, src2d, dst2d)
    g2 = _tc_mid(p, dis, b1.reshape(1, -1), W2)
    q = _sc_aggregate(g2, src2d, dst2d)
    out = _tc_final(q, dis, b2.reshape(1, -1), Wfc, bfc.reshape(1, -1))
    return out


# speedup vs baseline: 1.0861x; 1.0010x over previous
import functools

import jax
import jax.numpy as jnp
from jax import lax
from jax.experimental import pallas as pl
from jax.experimental.pallas import tpu as pltpu
from jax.experimental.pallas import tpu_sc as plsc

NC = 2
NS = 16
NW = NC * NS


def _worker_id():
    c = lax.axis_index("c")
    s = lax.axis_index("s")
    return c, s, s * NC + c


def _split_rows(n):
    span_a = ((n + NS - 1) // NS + 7) // 8 * 8
    span_last = n - span_a * (NS - 1)
    assert span_last > 0 and span_last % 8 == 0 and span_a % 8 == 0
    return span_a, span_last


def _per_tile_rows(s, n, fn):
    span_a, span_last = _split_rows(n)

    @pl.when(s < NS - 1)
    def _():
        fn(s * span_a, span_a)

    @pl.when(s == NS - 1)
    def _():
        fn((NS - 1) * span_a, span_last)


_DRING = 8
_DBLK = 8


def _sc_degree(dst2d, n_pad):
    nb_total, eba = dst2d.shape
    tpw = nb_total // NW
    nblk = tpw // _DBLK
    per_tile = n_pad // NS
    assert tpw * NW == nb_total and nblk * _DBLK == tpw and nblk % 2 == 0

    mesh = plsc.VectorSubcoreMesh(core_axis_name="c", subcore_axis_name="s")

    @functools.partial(
        pl.kernel,
        out_type=jax.ShapeDtypeStruct((NC * n_pad,), jnp.float32),
        mesh=mesh,
        scratch_types=[
            pltpu.VMEM((2, _DBLK, eba), jnp.int32),
            pltpu.VMEM((128,), jnp.float32),
            pltpu.VMEM((per_tile,), jnp.float32),
            pltpu.VMEM_SHARED((n_pad,), jnp.float32),
            pltpu.SemaphoreType.DMA((2,)),
            pltpu.SemaphoreType.DMA((_DRING,)),
        ],
    )
    def deg_kernel(dst_hbm, out_hbm, idx_v, ones_v, zbuf, acc, sem_i, sem_s):
        c, s, w = _worker_id()
        for j in range(128 // 16):
            ones_v[pl.ds(j * 16, 16)] = jnp.full((16,), 1.0, jnp.float32)
        for j in range(per_tile // 16):
            zbuf[pl.ds(j * 16, 16)] = jnp.zeros((16,), jnp.float32)

        def fire_idx(r, pb):
            pltpu.async_copy(dst_hbm.at[pl.ds(w * tpw + r * _DBLK, _DBLK)],
                             idx_v.at[pb], sem_i.at[pb])

        def drain_idx(pb):
            pltpu.make_async_copy(dst_hbm.at[pl.ds(0, _DBLK)], idx_v.at[pb],
                                  sem_i.at[pb]).wait()

        def fire_scatter(pb, row, j):
            pltpu.async_copy(ones_v.at[pl.ds(0, eba)],
                             acc.at[idx_v.at[pb].at[row]],
                             sem_s.at[j], add=True)

        def drain_scatter(j):
            pltpu.make_async_copy(ones_v.at[pl.ds(0, eba)],
                                  acc.at[idx_v.at[0].at[0]],
                                  sem_s.at[j]).wait()

        fire_idx(0, 0)
        pltpu.sync_copy(zbuf, acc.at[pl.ds(s * per_tile, per_tile)])
        plsc.subcore_barrier()

        def body(t, carry):
            for rr in range(2):
                r = t * 2 + rr
                for jj in range(_DBLK):
                    b = r * _DBLK + jj
                    if jj == 0:
                        drain_idx(rr)

                    @pl.when(b >= _DRING // 2)
                    def _():
                        drain_scatter((jj + _DRING // 2) % _DRING)

                    if jj == _DRING // 2:
                        @pl.when(r + 1 < nblk)
                        def _():
                            fire_idx(r + 1, 1 - rr)
                    fire_scatter(rr, jj, jj % _DRING)
            return carry

        lax.fori_loop(0, nblk // 2, body, 0)
        for k in range(_DRING // 2):
            drain_scatter((tpw - _DRING // 2 + k) % _DRING)
        plsc.subcore_barrier()
        pltpu.sync_copy(acc.at[pl.ds(s * per_tile, per_tile)],
                        out_hbm.at[pl.ds(c * n_pad + s * per_tile, per_tile)])

    return deg_kernel(dst2d)


_NBUF = 10
_LOOK = 5
_BLK = 20


def _sc_aggregate(g2h, src2d, dst2d):
    _, N, Dh = g2h.shape
    nb_total, eba = src2d.shape
    tpb = nb_total // NS
    nblk = tpb // _BLK
    assert tpb * NS == nb_total and nblk * _BLK == tpb and nblk % 2 == 0
    assert _BLK % _NBUF == 0 and _LOOK * 2 == _NBUF

    mesh = plsc.VectorSubcoreMesh(core_axis_name="c", subcore_axis_name="s")

    @functools.partial(
        pl.kernel,
        out_type=jax.ShapeDtypeStruct((NC, N, Dh), jnp.float32),
        mesh=mesh,
        scratch_types=[
            pltpu.VMEM((2, _BLK, eba), jnp.int32),
            pltpu.VMEM((2, _BLK, eba), jnp.int32),
            pltpu.VMEM((_NBUF, eba, Dh), jnp.float32),
            pltpu.VMEM_SHARED((N, Dh), jnp.float32),
            pltpu.SemaphoreType.DMA((_NBUF,)),
            pltpu.SemaphoreType.DMA((_NBUF,)),
            pltpu.SemaphoreType.DMA((2,)),
        ],
        compiler_params=pltpu.CompilerParams(use_tc_tiling_on_sc=False),
    )
    def agg_kernel(g_hbm, src_hbm, dst_hbm, out_hbm,
                   idx_s, idx_d, rows, acc, sem_g, sem_s, sem_i):
        c, s, _ = _worker_id()

        def on_half(fn):
            @pl.when(c == 0)
            def _():
                fn(g_hbm.at[0], out_hbm.at[0])

            @pl.when(c == 1)
            def _():
                fn(g_hbm.at[1], out_hbm.at[1])

        def fire_idx_block(r, pb):
            base = s * tpb + r * _BLK
            pltpu.async_copy(src_hbm.at[pl.ds(base, _BLK)], idx_s.at[pb],
                             sem_i.at[pb])
            pltpu.async_copy(dst_hbm.at[pl.ds(base, _BLK)], idx_d.at[pb],
                             sem_i.at[pb])

        def drain_idx_block(pb):
            pltpu.make_async_copy(src_hbm.at[pl.ds(0, _BLK)], idx_s.at[pb],
                                  sem_i.at[pb]).wait()
            pltpu.make_async_copy(dst_hbm.at[pl.ds(0, _BLK)], idx_d.at[pb],
                                  sem_i.at[pb]).wait()

        fire_idx_block(0, 0)

        def init(gh, oh):
            def cp(row0, span):
                pltpu.sync_copy(gh.at[pl.ds(row0, span)],
                                acc.at[pl.ds(row0, span)])
            _per_tile_rows(s, N, cp)

        on_half(init)
        drain_idx_block(0)
        plsc.subcore_barrier()

        def run_edges(gh, oh):
            def fire_gather(pb, row, j):
                pltpu.async_copy(gh.at[idx_s.at[pb].at[row]], rows.at[j],
                                 sem_g.at[j])

            def drain_gather(j):
                pltpu.make_async_copy(gh.at[idx_s.at[0].at[0]], rows.at[j],
                                      sem_g.at[j]).wait()

            def fire_scatter(pb, row, j):
                pltpu.async_copy(rows.at[j], acc.at[idx_d.at[pb].at[row]],
                                 sem_s.at[j], add=True)

            def drain_scatter(j):
                pltpu.make_async_copy(rows.at[j], acc.at[idx_d.at[0].at[0]],
                                      sem_s.at[j]).wait()

            for j in range(_LOOK):
                fire_gather(0, j, j)

            def body(t, carry):
                for rr in range(2):
                    r = t * 2 + rr
                    for jj in range(_BLK):
                        b = r * _BLK + jj
                        j = (b % _NBUF)
                        j2 = (jj + _LOOK) % _NBUF
                        drain_gather(j)
                        fire_scatter(rr, jj, j)

                        @pl.when(b >= _LOOK)
                        def _():
                            drain_scatter(j2)

                        if jj == _LOOK:
                            @pl.when(r + 1 < nblk)
                            def _():
                                fire_idx_block(r + 1, 1 - rr)
                        if jj == _BLK - _LOOK:
                            @pl.when(r + 1 < nblk)
                            def _():
                                drain_idx_block(1 - rr)
                        if jj < _BLK - _LOOK:
                            fire_gather(rr, jj + _LOOK, j2)
                        else:
                            @pl.when(b + _LOOK < tpb)
                            def _():
                                fire_gather(1 - rr, jj + _LOOK - _BLK, j2)
                return carry

            lax.fori_loop(0, nblk // 2, body, 0)
            for k in range(_LOOK):
                drain_scatter((tpb - _LOOK + k) % _NBUF)

        on_half(run_edges)
        plsc.subcore_barrier()

        def writeout(gh, oh):
            def cp(row0, span):
                pltpu.sync_copy(acc.at[pl.ds(row0, span)],
                                oh.at[pl.ds(row0, span)])
            _per_tile_rows(s, N, cp)

        on_half(writeout)

    return agg_kernel(g2h, src2d, dst2d)


_BN = 2000


def _tc_prologue(x, W1, degp3):
    N, D = x.shape
    H = W1.shape[0]
    Dh = H // 2

    def body(x_ref, w_ref, dp_ref, dis_ref, g_ref):
        deg = dp_ref[0] + dp_ref[1] + 1.0
        dis = lax.rsqrt(deg)
        dis_ref[...] = dis
        h = lax.dot_general(x_ref[...], w_ref[...],
                            (((1,), (1,)), ((), ())),
                            preferred_element_type=jnp.float32)
        v = h * dis
        g_ref[0] = v[:, :Dh]
        g_ref[1] = v[:, Dh:]

    return pl.pallas_call(
        body,
        grid=(N // _BN,),
        in_specs=[
            pl.BlockSpec((_BN, D), lambda i: (i, 0)),
            pl.BlockSpec((H, D), lambda i: (0, 0)),
            pl.BlockSpec((2, _BN, 1), lambda i: (0, i, 0)),
        ],
        out_specs=[
            pl.BlockSpec((_BN, 1), lambda i: (i, 0)),
            pl.BlockSpec((2, _BN, Dh), lambda i: (0, i, 0)),
        ],
        out_shape=[
            jax.ShapeDtypeStruct((N, 1), jnp.float32),
            jax.ShapeDtypeStruct((2, N, Dh), jnp.float32),
        ],
    )(x, W1, degp3)


def _tc_mid(p, dis, b, W):
    _, N, Dh = p.shape
    H = 2 * Dh
    Ho = W.shape[0]

    def body(p_ref, dis_ref, b_ref, w_ref, g_ref):
        agg = jnp.concatenate([p_ref[0], p_ref[1]], axis=1)
        z = jnp.maximum(dis_ref[...] * agg + b_ref[...], 0.0)
        h = lax.dot_general(z, w_ref[...], (((1,), (1,)), ((), ())),
                            preferred_element_type=jnp.float32)
        v = dis_ref[...] * h
        g_ref[0] = v[:, :Ho // 2]
        g_ref[1] = v[:, Ho // 2:]

    return pl.pallas_call(
        body,
        grid=(N // _BN,),
        in_specs=[
            pl.BlockSpec((2, _BN, Dh), lambda i: (0, i, 0)),
            pl.BlockSpec((_BN, 1), lambda i: (i, 0)),
            pl.BlockSpec((1, H), lambda i: (0, 0)),
            pl.BlockSpec((Ho, H), lambda i: (0, 0)),
        ],
        out_specs=pl.BlockSpec((2, _BN, Ho // 2), lambda i: (0, i, 0)),
        out_shape=jax.ShapeDtypeStruct((2, N, Ho // 2), jnp.float32),
    )(p, dis, b, W)


def _tc_final(q, dis, b, Wfc, bfc):
    _, N, Dh = q.shape
    H = 2 * Dh
    EMB = Wfc.shape[0]

    def body(q_ref, dis_ref, b_ref, w_ref, bfc_ref, out_ref):
        agg = jnp.concatenate([q_ref[0], q_ref[1]], axis=1)
        z = jnp.maximum(dis_ref[...] * agg + b_ref[...], 0.0)
        h = lax.dot_general(z, w_ref[...], (((1,), (1,)), ((), ())),
                            preferred_element_type=jnp.float32)
        out_ref[...] = h + bfc_ref[...]

    return pl.pallas_call(
        body,
        grid=(N // _BN,),
        in_specs=[
            pl.BlockSpec((2, _BN, Dh), lambda i: (0, i, 0)),
            pl.BlockSpec((_BN, 1), lambda i: (i, 0)),
            pl.BlockSpec((1, H), lambda i: (0, 0)),
            pl.BlockSpec((EMB, H), lambda i: (0, 0)),
            pl.BlockSpec((1, EMB), lambda i: (0, 0)),
        ],
        out_specs=pl.BlockSpec((_BN, EMB), lambda i: (i, 0)),
        out_shape=jax.ShapeDtypeStruct((N, EMB), jnp.float32),
    )(q, dis, b, Wfc, bfc)


def kernel(x, edge_index, W1, b1, W2, b2, Wfc, bfc):
    N, D = x.shape
    n_pad = 10240

    E = edge_index.shape[1]
    eba = 125
    src2d = edge_index[0].reshape(E // eba, eba)
    dst2d = edge_index[1].reshape(E // eba, eba)
    dst2d_deg = edge_index[1].reshape(E // 125, 125)

    degp = _sc_degree(dst2d_deg, n_pad)
    degp3 = degp.reshape(2, n_pad, 1)
    dis, g1 = _tc_prologue(x, W1, degp3)
    p = _sc_aggregate(g1, src2d, dst2d)
    g2 = _tc_mid(p, dis, b1.reshape(1, -1), W2)
    q = _sc_aggregate(g2, src2d, dst2d)
    out = _tc_final(q, dis, b2.reshape(1, -1), Wfc, bfc.reshape(1, -1))
    return out
